# baseline (speedup 1.0000x reference)
import numpy as np
import jax
import jax.numpy as jnp
from jax.experimental import pallas as pl
from jax.experimental.pallas import tpu as pltpu

B, G, D, S, NEG = 64, 7, 1280, 5, 16
CELLS = G * G
E = B * CELLS
EP = 3200
K = NEG + 1
BP = 448
NBLK = sum(6 - s for s in range(S))
NGRP = EP // 128
P_TOTAL = sum(B * (G - 1 - s) * G for s in range(S))

_BLOCK_S = np.repeat(np.arange(S), [6 - s for s in range(S)]).astype(np.int32)


def _cpc_kernel(sref, c_ref, w_hbm, enc_hbm, b_ref, idx_ref, out_ref,
                w_scr, enc_scr, pred_scr, scores_scr, sem_w, sem_e):
    g = pl.program_id(0)

    @pl.when(g == 0)
    def _():
        pltpu.make_async_copy(enc_hbm, enc_scr, sem_e).start()
        pltpu.make_async_copy(enc_scr, enc_scr, sem_e).wait()

    s_cur = sref[g]
    s_prev = sref[jnp.maximum(g - 1, 0)]

    @pl.when(jnp.logical_or(g == 0, s_cur != s_prev))
    def _():
        pltpu.make_async_copy(w_hbm.at[s_cur], w_scr, sem_w).start()
        pltpu.make_async_copy(w_scr, w_scr, sem_w).wait()

    pred = jax.lax.dot_general(c_ref[...], w_scr[...],
                               (((1,), (1,)), ((), ())),
                               preferred_element_type=jnp.float32)
    pred = pred + b_ref[0]
    pred_scr[...] = pred.astype(jnp.bfloat16)
    scores_scr[...] = jax.lax.dot_general(
        pred_scr[...], enc_scr[...], (((1,), (1,)), ((), ())),
        preferred_element_type=jnp.float32)
    idx = idx_ref[...]
    low = jnp.bitwise_and(idx, 127)
    grp = jnp.right_shift(idx, 7)
    dots = jnp.zeros((BP, K), jnp.float32)
    for gg in range(NGRP):
        sel = jnp.take_along_axis(scores_scr[:, gg * 128:(gg + 1) * 128],
                                  low, axis=1)
        dots = jnp.where(grp == gg, sel, dots)
    m = jnp.max(dots, axis=1, keepdims=True)
    ex = jnp.exp(dots - m)
    lse = m + jnp.log(jnp.sum(ex, axis=1, keepdims=True))
    pos = dots[:, 0:1]
    loss_rows = lse - pos
    maxneg = jnp.max(dots[:, 1:], axis=1, keepdims=True)
    corr_rows = (pos >= maxneg).astype(jnp.float32)
    loss_s = jnp.sum(loss_rows)
    corr_s = jnp.sum(corr_rows)
    lane = jax.lax.broadcasted_iota(jnp.int32, (1, 128), 1)
    out_ref[0] = (jnp.where(lane == 0, loss_s, 0.0)
                  + jnp.where(lane == 1, corr_s, 0.0))


def kernel(contexts, encodings, Wk_w, Wk_b, ctx_idx, cand_idx):
    del ctx_idx
    cb = contexts.astype(jnp.bfloat16).reshape(B, CELLS, D)
    c_all = jnp.concatenate(
        [cb[:, :(6 - s) * G].reshape(-1, D) for s in range(S)], axis=0)
    enc_bf = jnp.pad(encodings.reshape(E, D).astype(jnp.bfloat16),
                     ((0, EP - E), (0, 0)))
    w_bf = Wk_w.astype(jnp.bfloat16)
    bias3 = Wk_b.reshape(S, 1, D)

    grid_spec = pltpu.PrefetchScalarGridSpec(
        num_scalar_prefetch=1,
        grid=(NBLK,),
        in_specs=[
            pl.BlockSpec((BP, D), lambda g, s: (g, 0)),
            pl.BlockSpec(memory_space=pl.ANY),
            pl.BlockSpec(memory_space=pl.ANY),
            pl.BlockSpec((1, 1, D), lambda g, s: (s[g], 0, 0)),
            pl.BlockSpec((BP, K), lambda g, s: (g, 0)),
        ],
        out_specs=pl.BlockSpec((1, 1, 128), lambda g, s: (g, 0, 0)),
        scratch_shapes=[
            pltpu.VMEM((D, D), jnp.bfloat16),
            pltpu.VMEM((EP, D), jnp.bfloat16),
            pltpu.VMEM((BP, D), jnp.bfloat16),
            pltpu.VMEM((BP, EP), jnp.float32),
            pltpu.SemaphoreType.DMA,
            pltpu.SemaphoreType.DMA,
        ],
    )
    parts = pl.pallas_call(
        _cpc_kernel,
        grid_spec=grid_spec,
        out_shape=jax.ShapeDtypeStruct((NBLK, 1, 128), jnp.float32),
        compiler_params=pltpu.CompilerParams(
            dimension_semantics=("arbitrary",),
            vmem_limit_bytes=64 * 1024 * 1024,
        ),
    )(jnp.asarray(_BLOCK_S), c_all, w_bf, enc_bf, bias3, cand_idx)
    total = parts.sum(axis=(0, 1))
    return total[0] / P_TOTAL, total[1] / P_TOTAL
